# baseline (speedup 1.0000x reference)
import functools

import jax
import jax.numpy as jnp
from jax import lax
from jax.experimental import pallas as pl
from jax.experimental.pallas import tpu as pltpu
from jax.experimental.pallas import tpu_sc as plsc

N = 10000
E = 320000
DX = 128
DY = 40
D = 168
NPAD = 10240
ZROW = N
NC = 2
NS = 16
NT = NC * NS
EPT = E // NT

NBD = EPT // 16

EB = 20
NB = EPT // EB
W = 50
NW = NB // W

HR = NPAD // 16
RPT = NPAD // NS
ALPHA = 0.1

_SC_PARAMS = pltpu.CompilerParams(
    needs_layout_passes=False, use_tc_tiling_on_sc=False)


@functools.cache
def _get_deg_kernel():
    mesh = plsc.VectorSubcoreMesh(core_axis_name="c", subcore_axis_name="s")
    return functools.partial(
        pl.kernel,
        out_type=(
            jax.ShapeDtypeStruct((NC, HR, 16), jnp.float32),
            jax.ShapeDtypeStruct((NT, NBD, 16), jnp.int32),
        ),
        mesh=mesh,
        scratch_types=[
            pltpu.VMEM((NBD, 16), jnp.int32),
            pltpu.VMEM((NBD, 16), jnp.int32),
            pltpu.VMEM((NBD, 16), jnp.int32),
            pltpu.VMEM((HR, 16), jnp.float32),
            pltpu.VMEM((HR, 16), jnp.float32),
            pltpu.VMEM((HR // 128, 128), jnp.int32),
            pltpu.VMEM_SHARED((HR, 16), jnp.float32),
        ],
        compiler_params=_SC_PARAMS,
    )(_deg_body)


def _deg_body(src_hbm, dst_hbm, degp_hbm, dstp_hbm,
              src_v, dst_v, dstp_v, hist, zbuf, rowidx, shist):
    c = lax.axis_index("c")
    s = lax.axis_index("s")
    wid = c * NS + s

    pltpu.sync_copy(src_hbm.at[wid], src_v)
    pltpu.sync_copy(dst_hbm.at[wid], dst_v)

    zero16 = jnp.zeros((16,), jnp.float32)

    def _zero(i, _):
        hist[i] = zero16
        zbuf[i] = zero16
        return 0
    lax.fori_loop(0, HR, _zero, 0)

    iota16 = lax.iota(jnp.int32, 16)
    for b in range(HR // 128):
        for k in range(8):
            rowidx[b, pl.ds(k * 16, 16)] = iota16 + (b * 128 + k * 16)

    ones16 = jnp.ones((16,), jnp.float32)

    def _count(i, _):
        s16 = src_v[i]
        d16 = dst_v[i]
        kept = s16 != d16
        dstp_v[i] = jnp.where(kept, d16, ZROW)
        row = lax.shift_right_logical(d16, 4)
        col = jnp.bitwise_and(d16, 15)
        plsc.addupdate_scatter(hist, [row, col], ones16, mask=kept)
        return 0
    lax.fori_loop(0, NBD, _count, 0)

    pltpu.sync_copy(dstp_v, dstp_hbm.at[wid])

    @pl.when(s == 0)
    def _():
        pltpu.sync_copy(zbuf, shist)
    plsc.subcore_barrier()
    for b in range(HR // 128):
        pltpu.sync_copy(hist.at[pl.ds(b * 128, 128)],
                        shist.at[rowidx.at[b]], add=True)
    plsc.subcore_barrier()

    @pl.when(s == 0)
    def _():
        pltpu.sync_copy(shist, degp_hbm.at[c])


@functools.cache
def _get_hop_kernel():
    mesh = plsc.VectorSubcoreMesh(core_axis_name="c", subcore_axis_name="s")
    return functools.partial(
        pl.kernel,
        out_type=jax.ShapeDtypeStruct((NC, NPAD, D), jnp.float32),
        mesh=mesh,
        scratch_types=[
            pltpu.VMEM((W, EB), jnp.int32),
            pltpu.VMEM((W, EB), jnp.int32),
            pltpu.VMEM((W, EB), jnp.int32),
            pltpu.VMEM((W, EB), jnp.int32),
            pltpu.VMEM((EB, D), jnp.float32),
            pltpu.VMEM((EB, D), jnp.float32),
            pltpu.VMEM((EB, D), jnp.float32),
            pltpu.VMEM((EB, D), jnp.float32),
            pltpu.VMEM((EB, D), jnp.float32),
            pltpu.VMEM_SHARED((NPAD, D), jnp.float32),
        ] + [pltpu.SemaphoreType.DMA] * 12,
        compiler_params=_SC_PARAMS,
    )(_hop_body)


NBUF = 5


def _hop_body(mp_hbm, src_hbm, dstp_hbm, part_hbm,
              src_wA, dstp_wA, src_wB, dstp_wB, b0, b1, b2, b3, b4, acc,
              g0, g1, g2, g3, g4, s0, s1, s2, s3, s4, wsemA, wsemB):
    c = lax.axis_index("c")
    s = lax.axis_index("s")
    wid = c * NS + s
    bufs = [b0, b1, b2, b3, b4]
    gsem = [g0, g1, g2, g3, g4]
    ssem = [s0, s1, s2, s3, s4]

    nz = NPAD - N
    row0 = s * RPT
    done = 0
    while done < RPT:
        cnt = min(nz, RPT - done)
        pltpu.sync_copy(mp_hbm.at[pl.ds(ZROW, cnt)],
                        acc.at[pl.ds(row0 + done, cnt)])
        done += cnt
    plsc.subcore_barrier()

    def _sdrain(k, row, sw):
        pltpu.make_async_copy(bufs[k], acc.at[sw.at[row]], ssem[k]).wait()

    def _group(sw, dw, q, drain):
        rows = [NBUF * q + k for k in range(NBUF)]
        gds = []
        for k in range(NBUF):
            if drain:
                _sdrain(k, rows[k], sw)
            gds.append(pltpu.async_copy(
                mp_hbm.at[dw.at[rows[k]]], bufs[k], gsem[k]))
        for k in range(NBUF):
            gds[k].wait()
            pltpu.async_copy(bufs[k], acc.at[sw.at[rows[k]]],
                             ssem[k], add=True)

    pltpu.sync_copy(src_hbm.at[wid, pl.ds(0, W)], src_wA)
    pltpu.sync_copy(dstp_hbm.at[wid, pl.ds(0, W)], dstp_wA)

    for w in range(NW):
        if w % 2 == 0:
            sw, dw, nsw, ndw, nsem = src_wA, dstp_wA, src_wB, dstp_wB, wsemB
        else:
            sw, dw, nsw, ndw, nsem = src_wB, dstp_wB, src_wA, dstp_wA, wsemA

        _group(sw, dw, 0, drain=(w > 0))
        if w + 1 < NW:
            pltpu.async_copy(src_hbm.at[wid, pl.ds((w + 1) * W, W)],
                             nsw, nsem)
            pltpu.async_copy(dstp_hbm.at[wid, pl.ds((w + 1) * W, W)],
                             ndw, nsem)

        def _groupq(q, _, sw=sw, dw=dw):
            _group(sw, dw, q, drain=True)
            return 0
        lax.fori_loop(1, W // NBUF, _groupq, 0)

        if w + 1 < NW:
            pltpu.make_async_copy(src_hbm.at[wid, pl.ds((w + 1) * W, W)],
                                  nsw, nsem).wait()
            pltpu.make_async_copy(dstp_hbm.at[wid, pl.ds((w + 1) * W, W)],
                                  ndw, nsem).wait()

    last = src_wB if (NW - 1) % 2 else src_wA
    for k in range(NBUF):
        _sdrain(k, W - NBUF + k, last)

    plsc.subcore_barrier()
    pltpu.sync_copy(acc.at[pl.ds(s * RPT, RPT)],
                    part_hbm.at[c, pl.ds(s * RPT, RPT)])


_RB = 1280
_GRID = NPAD // _RB


def _prescale(degp, xp, yp):
    def body(degp_ref, x_ref, y_ref, dis_ref, mp_ref):
        deg = degp_ref[0] + degp_ref[1] + 1.0
        dis = lax.rsqrt(deg)
        dis_ref[...] = dis
        mp_ref[:, :DX] = x_ref[...] * dis
        mp_ref[:, DX:] = y_ref[...] * dis

    return pl.pallas_call(
        body,
        grid=(_GRID,),
        in_specs=[
            pl.BlockSpec((2, _RB, 1), lambda i: (0, i, 0)),
            pl.BlockSpec((_RB, DX), lambda i: (i, 0)),
            pl.BlockSpec((_RB, DY), lambda i: (i, 0)),
        ],
        out_specs=[
            pl.BlockSpec((_RB, 1), lambda i: (i, 0)),
            pl.BlockSpec((_RB, D), lambda i: (i, 0)),
        ],
        out_shape=[
            jax.ShapeDtypeStruct((NPAD, 1), jnp.float32),
            jax.ShapeDtypeStruct((NPAD, D), jnp.float32),
        ],
    )(degp, xp, yp)


def _combine1(part, mp, xp, yp, dis):
    def body(part_ref, mp_ref, x_ref, y_ref, dis_ref, m1_ref, mp1_ref):
        dis = dis_ref[...]
        h = dis * (part_ref[0] + part_ref[1] + mp_ref[...])
        hx = (1.0 - ALPHA) * h[:, :DX] + ALPHA * x_ref[...]
        hy = (1.0 - ALPHA) * h[:, DX:] + ALPHA * y_ref[...]
        m1_ref[:, :DX] = hx
        m1_ref[:, DX:] = hy
        mp1_ref[:, :DX] = dis * hx
        mp1_ref[:, DX:] = dis * hy

    return pl.pallas_call(
        body,
        grid=(_GRID,),
        in_specs=[
            pl.BlockSpec((2, _RB, D), lambda i: (0, i, 0)),
            pl.BlockSpec((_RB, D), lambda i: (i, 0)),
            pl.BlockSpec((_RB, DX), lambda i: (i, 0)),
            pl.BlockSpec((_RB, DY), lambda i: (i, 0)),
            pl.BlockSpec((_RB, 1), lambda i: (i, 0)),
        ],
        out_specs=[
            pl.BlockSpec((_RB, D), lambda i: (i, 0)),
            pl.BlockSpec((_RB, D), lambda i: (i, 0)),
        ],
        out_shape=[
            jax.ShapeDtypeStruct((NPAD, D), jnp.float32),
            jax.ShapeDtypeStruct((NPAD, D), jnp.float32),
        ],
    )(part, mp, xp, yp, dis)


_RB2 = 2000


def _combine2(part, mp, m, dis):
    def body(part_ref, mp_ref, m_ref, dis_ref, x_ref, y_ref):
        h = dis_ref[...] * (part_ref[0] + part_ref[1] + mp_ref[...])
        m2 = (1.0 - ALPHA) * h + ALPHA * m_ref[...]
        x_ref[...] = m2[:, :DX]
        y_ref[...] = m2[:, DX:]

    return pl.pallas_call(
        body,
        grid=(N // _RB2,),
        in_specs=[
            pl.BlockSpec((2, _RB2, D), lambda i: (0, i, 0)),
            pl.BlockSpec((_RB2, D), lambda i: (i, 0)),
            pl.BlockSpec((_RB2, D), lambda i: (i, 0)),
            pl.BlockSpec((_RB2, 1), lambda i: (i, 0)),
        ],
        out_specs=[
            pl.BlockSpec((_RB2, DX), lambda i: (i, 0)),
            pl.BlockSpec((_RB2, DY), lambda i: (i, 0)),
        ],
        out_shape=[
            jax.ShapeDtypeStruct((N, DX), jnp.float32),
            jax.ShapeDtypeStruct((N, DY), jnp.float32),
        ],
    )(part, mp, m, dis)


def kernel(x, y, edge_index):
    xp = jnp.pad(x, ((0, NPAD - N), (0, 0)))
    yp = jnp.pad(y, ((0, NPAD - N), (0, 0)))

    src = edge_index[0]
    dst = edge_index[1]

    degp, dstp = _get_deg_kernel()(src.reshape(NT, NBD, 16),
                                   dst.reshape(NT, NBD, 16))
    degp = degp.reshape(NC, NPAD, 1)

    dis, mp = _prescale(degp, xp, yp)

    src_b = src.reshape(NT, NB, EB)
    dstp_b = dstp.reshape(NT, NB, EB)

    part = _get_hop_kernel()(mp, src_b, dstp_b)
    m1, mp1 = _combine1(part, mp, xp, yp, dis)
    part2 = _get_hop_kernel()(mp1, src_b, dstp_b)
    return _combine2(part2, mp1, m1, dis)


# speedup vs baseline: 1.0217x; 1.0217x over previous
import functools

import jax
import jax.numpy as jnp
from jax import lax
from jax.experimental import pallas as pl
from jax.experimental.pallas import tpu as pltpu
from jax.experimental.pallas import tpu_sc as plsc

N = 10000
E = 320000
DX = 128
DY = 40
D = 168
NPAD = 10240
ZROW = N
NC = 2
NS = 16
NT = NC * NS
EPT = E // NT

NBD = EPT // 16

EB = 25
NB = EPT // EB
W = 40
NW = NB // W

HR = NPAD // 16
RPT = NPAD // NS
ALPHA = 0.1

_SC_PARAMS = pltpu.CompilerParams(
    needs_layout_passes=False, use_tc_tiling_on_sc=False)


@functools.cache
def _get_deg_kernel():
    mesh = plsc.VectorSubcoreMesh(core_axis_name="c", subcore_axis_name="s")
    return functools.partial(
        pl.kernel,
        out_type=(
            jax.ShapeDtypeStruct((NC, HR, 16), jnp.float32),
            jax.ShapeDtypeStruct((NT, NBD, 16), jnp.int32),
        ),
        mesh=mesh,
        scratch_types=[
            pltpu.VMEM((NBD, 16), jnp.int32),
            pltpu.VMEM((NBD, 16), jnp.int32),
            pltpu.VMEM((NBD, 16), jnp.int32),
            pltpu.VMEM((HR, 16), jnp.float32),
            pltpu.VMEM((HR, 16), jnp.float32),
            pltpu.VMEM((HR // 128, 128), jnp.int32),
            pltpu.VMEM_SHARED((HR, 16), jnp.float32),
        ],
        compiler_params=_SC_PARAMS,
    )(_deg_body)


def _deg_body(src_hbm, dst_hbm, degp_hbm, dstp_hbm,
              src_v, dst_v, dstp_v, hist, zbuf, rowidx, shist):
    c = lax.axis_index("c")
    s = lax.axis_index("s")
    wid = c * NS + s

    pltpu.sync_copy(src_hbm.at[wid], src_v)
    pltpu.sync_copy(dst_hbm.at[wid], dst_v)

    zero16 = jnp.zeros((16,), jnp.float32)

    def _zero(i, _):
        hist[i] = zero16
        zbuf[i] = zero16
        return 0
    lax.fori_loop(0, HR, _zero, 0)

    iota16 = lax.iota(jnp.int32, 16)
    for b in range(HR // 128):
        for k in range(8):
            rowidx[b, pl.ds(k * 16, 16)] = iota16 + (b * 128 + k * 16)

    ones16 = jnp.ones((16,), jnp.float32)

    def _count(i, _):
        s16 = src_v[i]
        d16 = dst_v[i]
        kept = s16 != d16
        dstp_v[i] = jnp.where(kept, d16, ZROW)
        row = lax.shift_right_logical(d16, 4)
        col = jnp.bitwise_and(d16, 15)
        plsc.addupdate_scatter(hist, [row, col], ones16, mask=kept)
        return 0
    lax.fori_loop(0, NBD, _count, 0)

    pltpu.sync_copy(dstp_v, dstp_hbm.at[wid])

    @pl.when(s == 0)
    def _():
        pltpu.sync_copy(zbuf, shist)
    plsc.subcore_barrier()
    for b in range(HR // 128):
        pltpu.sync_copy(hist.at[pl.ds(b * 128, 128)],
                        shist.at[rowidx.at[b]], add=True)
    plsc.subcore_barrier()

    @pl.when(s == 0)
    def _():
        pltpu.sync_copy(shist, degp_hbm.at[c])


@functools.cache
def _get_hop_kernel():
    mesh = plsc.VectorSubcoreMesh(core_axis_name="c", subcore_axis_name="s")
    return functools.partial(
        pl.kernel,
        out_type=jax.ShapeDtypeStruct((NC, NPAD, D), jnp.float32),
        mesh=mesh,
        scratch_types=[
            pltpu.VMEM((W, EB), jnp.int32),
            pltpu.VMEM((W, EB), jnp.int32),
            pltpu.VMEM((W, EB), jnp.int32),
            pltpu.VMEM((W, EB), jnp.int32),
            pltpu.VMEM((EB, D), jnp.float32),
            pltpu.VMEM((EB, D), jnp.float32),
            pltpu.VMEM((EB, D), jnp.float32),
            pltpu.VMEM((EB, D), jnp.float32),
            pltpu.VMEM_SHARED((NPAD, D), jnp.float32),
        ] + [pltpu.SemaphoreType.DMA] * 10,
        compiler_params=_SC_PARAMS,
    )(_hop_body)


NBUF = 4


def _hop_body(mp_hbm, src_hbm, dstp_hbm, part_hbm,
              src_wA, dstp_wA, src_wB, dstp_wB, b0, b1, b2, b3, acc,
              g0, g1, g2, g3, s0, s1, s2, s3, wsemA, wsemB):
    c = lax.axis_index("c")
    s = lax.axis_index("s")
    wid = c * NS + s
    bufs = [b0, b1, b2, b3]
    gsem = [g0, g1, g2, g3]
    ssem = [s0, s1, s2, s3]

    nz = NPAD - N
    row0 = s * RPT
    done = 0
    while done < RPT:
        cnt = min(nz, RPT - done)
        pltpu.sync_copy(mp_hbm.at[pl.ds(ZROW, cnt)],
                        acc.at[pl.ds(row0 + done, cnt)])
        done += cnt
    plsc.subcore_barrier()

    def _sdrain(k, row, sw):
        pltpu.make_async_copy(bufs[k], acc.at[sw.at[row]], ssem[k]).wait()

    def _group(sw, dw, q, drain):
        rows = [NBUF * q + k for k in range(NBUF)]
        gds = []
        for k in range(NBUF):
            if drain:
                _sdrain(k, rows[k], sw)
            gds.append(pltpu.async_copy(
                mp_hbm.at[dw.at[rows[k]]], bufs[k], gsem[k]))
        for k in range(NBUF):
            gds[k].wait()
            pltpu.async_copy(bufs[k], acc.at[sw.at[rows[k]]],
                             ssem[k], add=True)

    pltpu.sync_copy(src_hbm.at[wid, pl.ds(0, W)], src_wA)
    pltpu.sync_copy(dstp_hbm.at[wid, pl.ds(0, W)], dstp_wA)

    for w in range(NW):
        if w % 2 == 0:
            sw, dw, nsw, ndw, nsem = src_wA, dstp_wA, src_wB, dstp_wB, wsemB
        else:
            sw, dw, nsw, ndw, nsem = src_wB, dstp_wB, src_wA, dstp_wA, wsemA

        _group(sw, dw, 0, drain=(w > 0))
        if w + 1 < NW:
            pltpu.async_copy(src_hbm.at[wid, pl.ds((w + 1) * W, W)],
                             nsw, nsem)
            pltpu.async_copy(dstp_hbm.at[wid, pl.ds((w + 1) * W, W)],
                             ndw, nsem)

        def _groupq(q, _, sw=sw, dw=dw):
            _group(sw, dw, q, drain=True)
            return 0
        lax.fori_loop(1, W // NBUF, _groupq, 0)

        if w + 1 < NW:
            pltpu.make_async_copy(src_hbm.at[wid, pl.ds((w + 1) * W, W)],
                                  nsw, nsem).wait()
            pltpu.make_async_copy(dstp_hbm.at[wid, pl.ds((w + 1) * W, W)],
                                  ndw, nsem).wait()

    last = src_wB if (NW - 1) % 2 else src_wA
    for k in range(NBUF):
        _sdrain(k, W - NBUF + k, last)

    plsc.subcore_barrier()
    pltpu.sync_copy(acc.at[pl.ds(s * RPT, RPT)],
                    part_hbm.at[c, pl.ds(s * RPT, RPT)])


_RB = 1280
_GRID = NPAD // _RB


def _prescale(degp, xp, yp):
    def body(degp_ref, x_ref, y_ref, dis_ref, mp_ref):
        deg = degp_ref[0] + degp_ref[1] + 1.0
        dis = lax.rsqrt(deg)
        dis_ref[...] = dis
        mp_ref[:, :DX] = x_ref[...] * dis
        mp_ref[:, DX:] = y_ref[...] * dis

    return pl.pallas_call(
        body,
        grid=(_GRID,),
        in_specs=[
            pl.BlockSpec((2, _RB, 1), lambda i: (0, i, 0)),
            pl.BlockSpec((_RB, DX), lambda i: (i, 0)),
            pl.BlockSpec((_RB, DY), lambda i: (i, 0)),
        ],
        out_specs=[
            pl.BlockSpec((_RB, 1), lambda i: (i, 0)),
            pl.BlockSpec((_RB, D), lambda i: (i, 0)),
        ],
        out_shape=[
            jax.ShapeDtypeStruct((NPAD, 1), jnp.float32),
            jax.ShapeDtypeStruct((NPAD, D), jnp.float32),
        ],
    )(degp, xp, yp)


def _combine1(part, mp, dis):
    def body(part_ref, mp_ref, dis_ref, mp1_ref):
        dis = dis_ref[...]
        mp = mp_ref[...]
        h2 = (dis * dis) * (part_ref[0] + part_ref[1] + mp)
        mp1_ref[...] = (1.0 - ALPHA) * h2 + ALPHA * mp

    return pl.pallas_call(
        body,
        grid=(_GRID,),
        in_specs=[
            pl.BlockSpec((2, _RB, D), lambda i: (0, i, 0)),
            pl.BlockSpec((_RB, D), lambda i: (i, 0)),
            pl.BlockSpec((_RB, 1), lambda i: (i, 0)),
        ],
        out_specs=pl.BlockSpec((_RB, D), lambda i: (i, 0)),
        out_shape=jax.ShapeDtypeStruct((NPAD, D), jnp.float32),
    )(part, mp, dis)


_RB2 = 2000


def _combine2(part, mp1, dis):
    def body(part_ref, mp1_ref, dis_ref, x_ref, y_ref):
        dis = dis_ref[...]
        mp1 = mp1_ref[...]
        h = dis * (part_ref[0] + part_ref[1] + mp1)
        m2 = (1.0 - ALPHA) * h + ALPHA * (mp1 * (1.0 / dis))
        x_ref[...] = m2[:, :DX]
        y_ref[...] = m2[:, DX:]

    return pl.pallas_call(
        body,
        grid=(N // _RB2,),
        in_specs=[
            pl.BlockSpec((2, _RB2, D), lambda i: (0, i, 0)),
            pl.BlockSpec((_RB2, D), lambda i: (i, 0)),
            pl.BlockSpec((_RB2, 1), lambda i: (i, 0)),
        ],
        out_specs=[
            pl.BlockSpec((_RB2, DX), lambda i: (i, 0)),
            pl.BlockSpec((_RB2, DY), lambda i: (i, 0)),
        ],
        out_shape=[
            jax.ShapeDtypeStruct((N, DX), jnp.float32),
            jax.ShapeDtypeStruct((N, DY), jnp.float32),
        ],
    )(part, mp1, dis)


def kernel(x, y, edge_index):
    xp = jnp.pad(x, ((0, NPAD - N), (0, 0)))
    yp = jnp.pad(y, ((0, NPAD - N), (0, 0)))

    src = edge_index[0]
    dst = edge_index[1]

    degp, dstp = _get_deg_kernel()(src.reshape(NT, NBD, 16),
                                   dst.reshape(NT, NBD, 16))
    degp = degp.reshape(NC, NPAD, 1)

    dis, mp = _prescale(degp, xp, yp)

    src_b = src.reshape(NT, NB, EB)
    dstp_b = dstp.reshape(NT, NB, EB)

    part = _get_hop_kernel()(mp, src_b, dstp_b)
    mp1 = _combine1(part, mp, dis)
    part2 = _get_hop_kernel()(mp1, src_b, dstp_b)
    return _combine2(part2, mp1, dis)


# speedup vs baseline: 1.0229x; 1.0012x over previous
import functools

import jax
import jax.numpy as jnp
from jax import lax
from jax.experimental import pallas as pl
from jax.experimental.pallas import tpu as pltpu
from jax.experimental.pallas import tpu_sc as plsc

N = 10000
E = 320000
DX = 128
DY = 40
D = 168
NPAD = 10240
ZROW = N
NC = 2
NS = 16
NT = NC * NS
EPT = E // NT

NBD = EPT // 16

EB = 25
NB = EPT // EB
W = 40
NW = NB // W

HR = NPAD // 16
RPT = NPAD // NS
ALPHA = 0.1

_SC_PARAMS = pltpu.CompilerParams(
    needs_layout_passes=False, use_tc_tiling_on_sc=False)


@functools.cache
def _get_deg_kernel():
    mesh = plsc.VectorSubcoreMesh(core_axis_name="c", subcore_axis_name="s")
    return functools.partial(
        pl.kernel,
        out_type=(
            jax.ShapeDtypeStruct((NC, HR, 16), jnp.float32),
            jax.ShapeDtypeStruct((NT, NBD, 16), jnp.int32),
        ),
        mesh=mesh,
        scratch_types=[
            pltpu.VMEM((NBD, 16), jnp.int32),
            pltpu.VMEM((NBD, 16), jnp.int32),
            pltpu.VMEM((NBD, 16), jnp.int32),
            pltpu.VMEM((HR, 16), jnp.float32),
            pltpu.VMEM((HR, 16), jnp.float32),
            pltpu.VMEM((HR // 128, 128), jnp.int32),
            pltpu.VMEM_SHARED((HR, 16), jnp.float32),
        ],
        compiler_params=_SC_PARAMS,
    )(_deg_body)


def _deg_body(src_hbm, dst_hbm, degp_hbm, dstp_hbm,
              src_v, dst_v, dstp_v, hist, zbuf, rowidx, shist):
    c = lax.axis_index("c")
    s = lax.axis_index("s")
    wid = c * NS + s

    pltpu.sync_copy(src_hbm.at[wid], src_v)
    pltpu.sync_copy(dst_hbm.at[wid], dst_v)

    zero16 = jnp.zeros((16,), jnp.float32)

    def _zero(i, _):
        hist[i] = zero16
        zbuf[i] = zero16
        return 0
    lax.fori_loop(0, HR, _zero, 0)

    iota16 = lax.iota(jnp.int32, 16)
    for b in range(HR // 128):
        for k in range(8):
            rowidx[b, pl.ds(k * 16, 16)] = iota16 + (b * 128 + k * 16)

    ones16 = jnp.ones((16,), jnp.float32)

    def _count(i, _):
        s16 = src_v[i]
        d16 = dst_v[i]
        kept = s16 != d16
        dstp_v[i] = jnp.where(kept, d16, ZROW)
        row = lax.shift_right_logical(d16, 4)
        col = jnp.bitwise_and(d16, 15)
        plsc.addupdate_scatter(hist, [row, col], ones16, mask=kept)
        return 0
    lax.fori_loop(0, NBD, _count, 0)

    pltpu.sync_copy(dstp_v, dstp_hbm.at[wid])

    @pl.when(s == 0)
    def _():
        pltpu.sync_copy(zbuf, shist)
    plsc.subcore_barrier()
    for b in range(HR // 128):
        pltpu.sync_copy(hist.at[pl.ds(b * 128, 128)],
                        shist.at[rowidx.at[b]], add=True)
    plsc.subcore_barrier()

    @pl.when(s == 0)
    def _():
        pltpu.sync_copy(shist, degp_hbm.at[c])


@functools.cache
def _get_hop_kernel():
    mesh = plsc.VectorSubcoreMesh(core_axis_name="c", subcore_axis_name="s")
    return functools.partial(
        pl.kernel,
        out_type=jax.ShapeDtypeStruct((NC, NPAD, D), jnp.float32),
        mesh=mesh,
        scratch_types=[
            pltpu.VMEM((W, EB), jnp.int32),
            pltpu.VMEM((W, EB), jnp.int32),
            pltpu.VMEM((W, EB), jnp.int32),
            pltpu.VMEM((W, EB), jnp.int32),
            pltpu.VMEM((EB, D), jnp.float32),
            pltpu.VMEM((EB, D), jnp.float32),
            pltpu.VMEM((EB, D), jnp.float32),
            pltpu.VMEM((EB, D), jnp.float32),
            pltpu.VMEM_SHARED((NPAD, D), jnp.float32),
        ] + [pltpu.SemaphoreType.DMA] * 10,
        compiler_params=_SC_PARAMS,
    )(_hop_body)


NBUF = 4


def _hop_body(mp_hbm, src_hbm, dstp_hbm, part_hbm,
              src_wA, dstp_wA, src_wB, dstp_wB, b0, b1, b2, b3, acc,
              g0, g1, g2, g3, s0, s1, s2, s3, wsemA, wsemB):
    c = lax.axis_index("c")
    s = lax.axis_index("s")
    wid = c * NS + s
    bufs = [b0, b1, b2, b3]
    gsem = [g0, g1, g2, g3]
    ssem = [s0, s1, s2, s3]

    nz = NPAD - N
    row0 = s * RPT
    done = 0
    while done < RPT:
        cnt = min(nz, RPT - done)
        pltpu.sync_copy(mp_hbm.at[pl.ds(ZROW, cnt)],
                        acc.at[pl.ds(row0 + done, cnt)])
        done += cnt
    plsc.subcore_barrier()

    def _sdrain(k, row, sw):
        pltpu.make_async_copy(bufs[k], acc.at[sw.at[row]], ssem[k]).wait()

    def _group(sw, dw, q, drain):
        rows = [NBUF * q + k for k in range(NBUF)]
        gds = []
        for k in range(NBUF):
            if drain:
                _sdrain(k, rows[k], sw)
            gds.append(pltpu.async_copy(
                mp_hbm.at[dw.at[rows[k]]], bufs[k], gsem[k]))
        for k in range(NBUF):
            gds[k].wait()
            pltpu.async_copy(bufs[k], acc.at[sw.at[rows[k]]],
                             ssem[k], add=True)

    pltpu.sync_copy(src_hbm.at[wid, pl.ds(0, W)], src_wA)
    pltpu.sync_copy(dstp_hbm.at[wid, pl.ds(0, W)], dstp_wA)

    for w in range(NW):
        if w % 2 == 0:
            sw, dw, nsw, ndw, nsem = src_wA, dstp_wA, src_wB, dstp_wB, wsemB
        else:
            sw, dw, nsw, ndw, nsem = src_wB, dstp_wB, src_wA, dstp_wA, wsemA

        _group(sw, dw, 0, drain=(w > 0))
        if w + 1 < NW:
            pltpu.async_copy(src_hbm.at[wid, pl.ds((w + 1) * W, W)],
                             nsw, nsem)
            pltpu.async_copy(dstp_hbm.at[wid, pl.ds((w + 1) * W, W)],
                             ndw, nsem)

        def _groupq(q, _, sw=sw, dw=dw):
            _group(sw, dw, q, drain=True)
            return 0
        lax.fori_loop(1, W // NBUF, _groupq, 0)

        if w + 1 < NW:
            pltpu.make_async_copy(src_hbm.at[wid, pl.ds((w + 1) * W, W)],
                                  nsw, nsem).wait()
            pltpu.make_async_copy(dstp_hbm.at[wid, pl.ds((w + 1) * W, W)],
                                  ndw, nsem).wait()

    last = src_wB if (NW - 1) % 2 else src_wA
    for k in range(NBUF):
        _sdrain(k, W - NBUF + k, last)

    plsc.subcore_barrier()
    pltpu.sync_copy(acc.at[pl.ds(s * RPT, RPT)],
                    part_hbm.at[c, pl.ds(s * RPT, RPT)])


_RB = 2560
_GRID = NPAD // _RB


def _prescale(degp, xp, yp):
    def body(degp_ref, x_ref, y_ref, dis_ref, mp_ref):
        deg = degp_ref[0] + degp_ref[1] + 1.0
        dis = lax.rsqrt(deg)
        dis_ref[...] = dis
        mp_ref[:, :DX] = x_ref[...] * dis
        mp_ref[:, DX:] = y_ref[...] * dis

    return pl.pallas_call(
        body,
        grid=(_GRID,),
        in_specs=[
            pl.BlockSpec((2, _RB, 1), lambda i: (0, i, 0)),
            pl.BlockSpec((_RB, DX), lambda i: (i, 0)),
            pl.BlockSpec((_RB, DY), lambda i: (i, 0)),
        ],
        out_specs=[
            pl.BlockSpec((_RB, 1), lambda i: (i, 0)),
            pl.BlockSpec((_RB, D), lambda i: (i, 0)),
        ],
        out_shape=[
            jax.ShapeDtypeStruct((NPAD, 1), jnp.float32),
            jax.ShapeDtypeStruct((NPAD, D), jnp.float32),
        ],
    )(degp, xp, yp)


def _combine1(part, mp, dis):
    def body(part_ref, mp_ref, dis_ref, mp1_ref):
        dis = dis_ref[...]
        mp = mp_ref[...]
        h2 = (dis * dis) * (part_ref[0] + part_ref[1] + mp)
        mp1_ref[...] = (1.0 - ALPHA) * h2 + ALPHA * mp

    return pl.pallas_call(
        body,
        grid=(_GRID,),
        in_specs=[
            pl.BlockSpec((2, _RB, D), lambda i: (0, i, 0)),
            pl.BlockSpec((_RB, D), lambda i: (i, 0)),
            pl.BlockSpec((_RB, 1), lambda i: (i, 0)),
        ],
        out_specs=pl.BlockSpec((_RB, D), lambda i: (i, 0)),
        out_shape=jax.ShapeDtypeStruct((NPAD, D), jnp.float32),
    )(part, mp, dis)


_RB2 = 2000


def _combine2(part, mp1, dis):
    def body(part_ref, mp1_ref, dis_ref, x_ref, y_ref):
        dis = dis_ref[...]
        mp1 = mp1_ref[...]
        h = dis * (part_ref[0] + part_ref[1] + mp1)
        m2 = (1.0 - ALPHA) * h + ALPHA * (mp1 * (1.0 / dis))
        x_ref[...] = m2[:, :DX]
        y_ref[...] = m2[:, DX:]

    return pl.pallas_call(
        body,
        grid=(N // _RB2,),
        in_specs=[
            pl.BlockSpec((2, _RB2, D), lambda i: (0, i, 0)),
            pl.BlockSpec((_RB2, D), lambda i: (i, 0)),
            pl.BlockSpec((_RB2, 1), lambda i: (i, 0)),
        ],
        out_specs=[
            pl.BlockSpec((_RB2, DX), lambda i: (i, 0)),
            pl.BlockSpec((_RB2, DY), lambda i: (i, 0)),
        ],
        out_shape=[
            jax.ShapeDtypeStruct((N, DX), jnp.float32),
            jax.ShapeDtypeStruct((N, DY), jnp.float32),
        ],
    )(part, mp1, dis)


def kernel(x, y, edge_index):
    xp = jnp.pad(x, ((0, NPAD - N), (0, 0)))
    yp = jnp.pad(y, ((0, NPAD - N), (0, 0)))

    src = edge_index[0]
    dst = edge_index[1]

    degp, dstp = _get_deg_kernel()(src.reshape(NT, NBD, 16),
                                   dst.reshape(NT, NBD, 16))
    degp = degp.reshape(NC, NPAD, 1)

    dis, mp = _prescale(degp, xp, yp)

    src_b = src.reshape(NT, NB, EB)
    dstp_b = dstp.reshape(NT, NB, EB)

    part = _get_hop_kernel()(mp, src_b, dstp_b)
    mp1 = _combine1(part, mp, dis)
    part2 = _get_hop_kernel()(mp1, src_b, dstp_b)
    return _combine2(part2, mp1, dis)
